# baseline (speedup 1.0000x reference)
import functools

import jax
import jax.numpy as jnp
from jax import lax
from jax.experimental import pallas as pl
from jax.experimental.pallas import tpu as pltpu
from jax.experimental.pallas import tpu_sc as plsc

B = 1024
S = 200
D = 128
K = 50
KP = 64

NC = 2
NS = 16
NW = NC * NS
NSPLIT = 2
NBAT = B // NSPLIT
BPW = NBAT // NW
NBUF = 8
L = 16

_mesh = plsc.VectorSubcoreMesh(core_axis_name="c", subcore_axis_name="s")


def _make_gather(global_base):
    @functools.partial(
        pl.kernel,
        mesh=_mesh,
        out_type=jax.ShapeDtypeStruct((NBAT, K, D), jnp.float32),
        scratch_types=[
            pltpu.VMEM((KP,), jnp.int32),
            pltpu.VMEM((BPW, KP), jnp.int32),
            pltpu.VMEM((NBUF, K, D), jnp.float32),
            pltpu.SemaphoreType.DMA,
            pltpu.SemaphoreType.DMA,
        ],
    )
    def _gather(x_hbm, idx_hbm, out_hbm, idx_v, gidx, buf, gsem, wsem):
        wid = lax.axis_index("s") * NC + lax.axis_index("c")
        base_batch = wid * BPW

        pltpu.sync_copy(idx_hbm, idx_v)

        for v in range(KP // L):
            iv = idx_v[pl.ds(v * L, L)]
            for i in range(BPW):
                off = jnp.full(
                    (L,), (global_base + base_batch + i) * S, jnp.int32)
                gidx[i, pl.ds(v * L, L)] = iv + off

        def gstart(i):
            return pltpu.async_copy(
                x_hbm.at[gidx.at[i, pl.ds(0, K)]], buf.at[i % NBUF], gsem)

        gh = [None] * BPW
        wh = [None] * BPW
        for i in range(NBUF):
            gh[i] = gstart(i)
        for i in range(BPW):
            gh[i].wait()
            wh[i] = pltpu.async_copy(
                buf.at[i % NBUF], out_hbm.at[base_batch + i], wsem)
            ni = i + NBUF
            if ni < BPW:
                wh[i].wait()
                gh[ni] = gstart(ni)
        for i in range(BPW - NBUF, BPW):
            wh[i].wait()

    return _gather


_gathers = [_make_gather(s * NBAT) for s in range(NSPLIT)]


def kernel(x, index):
    x2d = x.reshape(B * S, D)
    idx_pad = jnp.zeros((KP,), jnp.int32).at[:K].set(index)
    parts = [g(x2d, idx_pad) for g in _gathers]
    return jnp.concatenate(parts, axis=0)


# speedup vs baseline: 2.2892x; 2.2892x over previous
import functools

import jax
import jax.numpy as jnp
from jax import lax
from jax.experimental import pallas as pl
from jax.experimental.pallas import tpu as pltpu
from jax.experimental.pallas import tpu_sc as plsc

B = 1024
S = 200
D = 128
K = 50
KP = 64

NC = 2
NS = 16
NW = NC * NS
BPW = B // NW
NBUF = 8
L = 16

_mesh = plsc.VectorSubcoreMesh(core_axis_name="c", subcore_axis_name="s")


@functools.partial(
    pl.kernel,
    mesh=_mesh,
    out_type=jax.ShapeDtypeStruct((K * B, D), jnp.float32),
    scratch_types=[
        pltpu.VMEM((KP,), jnp.int32),
        pltpu.VMEM((BPW, KP), jnp.int32),
        pltpu.VMEM((BPW, KP), jnp.int32),
        pltpu.VMEM((NBUF, K, D), jnp.float32),
        pltpu.SemaphoreType.DMA,
        pltpu.SemaphoreType.DMA,
    ],
)
def _gather(x_hbm, idx_hbm, out_hbm, idx_v, gidx, oidx, buf, gsem, wsem):
    wid = lax.axis_index("s") * NC + lax.axis_index("c")
    base_batch = wid * BPW

    pltpu.sync_copy(idx_hbm, idx_v)

    iota = lax.iota(jnp.int32, L)
    for v in range(KP // L):
        iv = idx_v[pl.ds(v * L, L)]
        kvec = iota + v * L
        for i in range(BPW):
            off = jnp.full((L,), (base_batch + i) * S, jnp.int32)
            gidx[i, pl.ds(v * L, L)] = iv + off
            ob = jnp.full((L,), base_batch + i, jnp.int32)
            oidx[i, pl.ds(v * L, L)] = kvec * B + ob

    def gstart(i):
        return pltpu.async_copy(
            x_hbm.at[gidx.at[i, pl.ds(0, K)]], buf.at[i % NBUF], gsem)

    gh = [None] * BPW
    wh = [None] * BPW
    for i in range(NBUF):
        gh[i] = gstart(i)
    for i in range(BPW):
        gh[i].wait()
        wh[i] = pltpu.async_copy(
            buf.at[i % NBUF], out_hbm.at[oidx.at[i, pl.ds(0, K)]], wsem)
        ni = i + NBUF
        if ni < BPW:
            wh[i].wait()
            gh[ni] = gstart(ni)
    for i in range(BPW - NBUF, BPW):
        wh[i].wait()


def kernel(x, index):
    x2d = x.reshape(B * S, D)
    idx_pad = jnp.zeros((KP,), jnp.int32).at[:K].set(index)
    out2d = _gather(x2d, idx_pad)
    return out2d.reshape(K, B, D).transpose(1, 0, 2)


# speedup vs baseline: 2.3008x; 1.0050x over previous
import functools

import jax
import jax.numpy as jnp
from jax import lax
from jax.experimental import pallas as pl
from jax.experimental.pallas import tpu as pltpu
from jax.experimental.pallas import tpu_sc as plsc

B = 1024
S = 200
D = 128
K = 50
KP = 64

NC = 2
NS = 16
NW = NC * NS
BPW = B // NW
NBUF = 12
L = 16

_mesh = plsc.VectorSubcoreMesh(core_axis_name="c", subcore_axis_name="s")


@functools.partial(
    pl.kernel,
    mesh=_mesh,
    out_type=jax.ShapeDtypeStruct((K * B, D), jnp.float32),
    scratch_types=[
        pltpu.VMEM((KP,), jnp.int32),
        pltpu.VMEM((BPW, KP), jnp.int32),
        pltpu.VMEM((BPW, KP), jnp.int32),
        pltpu.VMEM((NBUF, K, D), jnp.float32),
        pltpu.SemaphoreType.DMA,
        pltpu.SemaphoreType.DMA,
    ],
)
def _gather(x_hbm, idx_hbm, out_hbm, idx_v, gidx, oidx, buf, gsem, wsem):
    wid = lax.axis_index("s") * NC + lax.axis_index("c")
    base_batch = wid * BPW

    zeros = jnp.zeros((L,), jnp.int32)
    for v in range(KP // L):
        idx_v[pl.ds(v * L, L)] = zeros
    pltpu.sync_copy(idx_hbm, idx_v.at[pl.ds(0, K)])

    iota = lax.iota(jnp.int32, L)
    for v in range(KP // L):
        iv = idx_v[pl.ds(v * L, L)]
        kvec = iota + v * L
        for i in range(BPW):
            off = jnp.full((L,), (base_batch + i) * S, jnp.int32)
            gidx[i, pl.ds(v * L, L)] = iv + off
            ob = jnp.full((L,), base_batch + i, jnp.int32)
            oidx[i, pl.ds(v * L, L)] = kvec * B + ob

    def gstart(i):
        return pltpu.async_copy(
            x_hbm.at[gidx.at[i, pl.ds(0, K)]], buf.at[i % NBUF], gsem)

    gh = [None] * BPW
    wh = [None] * BPW
    for i in range(NBUF):
        gh[i] = gstart(i)
    for i in range(BPW):
        gh[i].wait()
        wh[i] = pltpu.async_copy(
            buf.at[i % NBUF], out_hbm.at[oidx.at[i, pl.ds(0, K)]], wsem)
        ni = i + NBUF
        if ni < BPW:
            wh[i].wait()
            gh[ni] = gstart(ni)
    for i in range(BPW - NBUF, BPW):
        wh[i].wait()


def kernel(x, index):
    x2d = x.reshape(B * S, D)
    out2d = _gather(x2d, index)
    return out2d.reshape(K, B, D).transpose(1, 0, 2)


# speedup vs baseline: 2.3113x; 1.0046x over previous
import functools

import jax
import jax.numpy as jnp
from jax import lax
from jax.experimental import pallas as pl
from jax.experimental.pallas import tpu as pltpu
from jax.experimental.pallas import tpu_sc as plsc

B = 1024
S = 200
D = 128
K = 50
K2 = 2 * K
K2P = 112

NC = 2
NS = 16
NW = NC * NS
BPW = B // NW
NCHK = BPW // 2
NBUF = 8
L = 16
_STARTS = (0, 16, 32, 48, 64, 80, 84)

_mesh = plsc.VectorSubcoreMesh(core_axis_name="c", subcore_axis_name="s")


@functools.partial(
    pl.kernel,
    mesh=_mesh,
    out_type=jax.ShapeDtypeStruct((K * B, D), jnp.float32),
    scratch_types=[
        pltpu.VMEM((K2P,), jnp.int32),
        pltpu.VMEM((NCHK, K2), jnp.int32),
        pltpu.VMEM((NCHK, K2), jnp.int32),
        pltpu.VMEM((NBUF, K2, D), jnp.float32),
        pltpu.SemaphoreType.DMA,
        pltpu.SemaphoreType.DMA,
    ],
)
def _gather(x_hbm, idx2_hbm, out_hbm, idx_v, gidx, oidx, buf, gsem, wsem):
    wid = lax.axis_index("s") * NC + lax.axis_index("c")
    base_batch = wid * BPW

    zeros = jnp.zeros((L,), jnp.int32)
    for v in range(K2P // L):
        idx_v[pl.ds(v * L, L)] = zeros
    pltpu.sync_copy(idx2_hbm, idx_v.at[pl.ds(0, K2)])

    iota = lax.iota(jnp.int32, L)
    fifty = jnp.full((L,), K, jnp.int32)
    for p in range(NCHK):
        b0 = base_batch + 2 * p
        b1 = b0 + 1
        for start in _STARTS:
            jv = iota + start
            sel = jv >= fifty
            dv = idx_v[pl.ds(start, L)]
            goff = jnp.where(sel, jnp.full((L,), b1 * S, jnp.int32),
                             jnp.full((L,), b0 * S, jnp.int32))
            gidx[p, pl.ds(start, L)] = dv + goff
            kv = jnp.where(sel, jv - fifty, jv)
            ob = jnp.where(sel, jnp.full((L,), b1, jnp.int32),
                           jnp.full((L,), b0, jnp.int32))
            oidx[p, pl.ds(start, L)] = kv * B + ob

    def gstart(p):
        return pltpu.async_copy(x_hbm.at[gidx.at[p]], buf.at[p % NBUF], gsem)

    gh = [None] * NCHK
    wh = [None] * NCHK
    for p in range(NBUF):
        gh[p] = gstart(p)
    for p in range(NCHK):
        gh[p].wait()
        wh[p] = pltpu.async_copy(
            buf.at[p % NBUF], out_hbm.at[oidx.at[p]], wsem)
        np_ = p + NBUF
        if np_ < NCHK:
            wh[p].wait()
            gh[np_] = gstart(np_)
    for p in range(NCHK - NBUF, NCHK):
        wh[p].wait()


def kernel(x, index):
    x2d = x.reshape(B * S, D)
    idx2 = jnp.concatenate([index, index])
    out2d = _gather(x2d, idx2)
    return out2d.reshape(K, B, D).transpose(1, 0, 2)


# speedup vs baseline: 2.3117x; 1.0002x over previous
import functools

import jax
import jax.numpy as jnp
from jax import lax
from jax.experimental import pallas as pl
from jax.experimental.pallas import tpu as pltpu
from jax.experimental.pallas import tpu_sc as plsc

B = 1024
S = 200
D = 128
K = 50
K2 = 2 * K
K2P = 112

NC = 2
NS = 16
NW = NC * NS
BPW = B // NW
NCHK = BPW // 2
NBUF = 8
L = 16
_STARTS = (0, 16, 32, 48, 64, 80, 84)

_mesh = plsc.VectorSubcoreMesh(core_axis_name="c", subcore_axis_name="s")


@functools.partial(
    pl.kernel,
    mesh=_mesh,
    out_type=jax.ShapeDtypeStruct((K * B, D), jnp.float32),
    scratch_types=[
        pltpu.VMEM((K2P,), jnp.int32),
        pltpu.VMEM((NCHK, K2), jnp.int32),
        pltpu.VMEM((NCHK, K2), jnp.int32),
        pltpu.VMEM((NBUF, K2, D), jnp.float32),
        pltpu.SemaphoreType.DMA,
        pltpu.SemaphoreType.DMA,
    ],
)
def _gather(x_hbm, idx2_hbm, out_hbm, idx_v, gidx, oidx, buf, gsem, wsem):
    wid = lax.axis_index("s") * NC + lax.axis_index("c")
    base_batch = wid * BPW

    pltpu.sync_copy(idx2_hbm, idx_v.at[pl.ds(0, K2)])

    iota = lax.iota(jnp.int32, L)
    fifty = jnp.full((L,), K, jnp.int32)

    def build_row(p):
        b0 = base_batch + 2 * p
        b1 = b0 + 1
        for start in _STARTS:
            jv = iota + start
            sel = jv >= fifty
            dv = idx_v[pl.ds(start, L)]
            goff = jnp.where(sel, jnp.full((L,), b1 * S, jnp.int32),
                             jnp.full((L,), b0 * S, jnp.int32))
            gidx[p, pl.ds(start, L)] = dv + goff
            kv = jnp.where(sel, jv - fifty, jv)
            ob = jnp.where(sel, jnp.full((L,), b1, jnp.int32),
                           jnp.full((L,), b0, jnp.int32))
            oidx[p, pl.ds(start, L)] = kv * B + ob

    def gstart(p):
        build_row(p)
        return pltpu.async_copy(x_hbm.at[gidx.at[p]], buf.at[p % NBUF], gsem)

    gh = [None] * NCHK
    wh = [None] * NCHK
    for p in range(NBUF):
        gh[p] = gstart(p)
    for p in range(NCHK):
        gh[p].wait()
        wh[p] = pltpu.async_copy(
            buf.at[p % NBUF], out_hbm.at[oidx.at[p]], wsem)
        np_ = p + NBUF
        if np_ < NCHK:
            wh[p].wait()
            gh[np_] = gstart(np_)
    for p in range(NCHK - NBUF, NCHK):
        wh[p].wait()


def kernel(x, index):
    x2d = x.reshape(B * S, D)
    idx2 = jnp.concatenate([index, index])
    out2d = _gather(x2d, idx2)
    return out2d.reshape(K, B, D).transpose(1, 0, 2)


# speedup vs baseline: 2.3376x; 1.0112x over previous
import functools

import jax
import jax.numpy as jnp
from jax import lax
from jax.experimental import pallas as pl
from jax.experimental.pallas import tpu as pltpu
from jax.experimental.pallas import tpu_sc as plsc

B = 1024
S = 200
D = 128
K = 50

NC = 2
NS = 16
NW = NC * NS
NWK = K // 2
CH = 128
NCHB = B // CH
NCHK = 2 * NCHB
NBUF = 4
L = 16

_mesh = plsc.VectorSubcoreMesh(core_axis_name="c", subcore_axis_name="s")


@functools.partial(
    pl.kernel,
    mesh=_mesh,
    out_type=jax.ShapeDtypeStruct((K * B, D), jnp.float32),
    scratch_types=[
        pltpu.VMEM((K * L,), jnp.int32),
        pltpu.VMEM((NCHK, CH), jnp.int32),
        pltpu.VMEM((NBUF, CH, D), jnp.float32),
        pltpu.SemaphoreType.DMA,
        pltpu.SemaphoreType.DMA,
    ],
)
def _gather(x_hbm, idxr_hbm, out_hbm, idx_v, gidx, buf, gsem, wsem):
    wid = lax.axis_index("s") * NC + lax.axis_index("c")

    @pl.when(wid < NWK)
    def _body():
        k0 = wid * 2
        pltpu.sync_copy(idxr_hbm, idx_v)

        iota = lax.iota(jnp.int32, L)

        def build_row(c):
            k = k0 + c // NCHB
            rep = idx_v[pl.ds(k * L, L)]
            b_base = (c % NCHB) * CH
            for h in range(CH // L):
                bv = jnp.full((L,), b_base + h * L, jnp.int32) + iota
                gidx[c, pl.ds(h * L, L)] = bv * S + rep

        def gstart(c):
            build_row(c)
            return pltpu.async_copy(
                x_hbm.at[gidx.at[c]], buf.at[c % NBUF], gsem)

        def wtarget(c):
            k = k0 + c // NCHB
            return out_hbm.at[pl.ds(k * B + (c % NCHB) * CH, CH)]

        gh = [None] * NCHK
        wh = [None] * NCHK
        for c in range(NBUF):
            gh[c] = gstart(c)
        for c in range(NCHK):
            gh[c].wait()
            wh[c] = pltpu.async_copy(buf.at[c % NBUF], wtarget(c), wsem)
            nc_ = c + NBUF
            if nc_ < NCHK:
                wh[c].wait()
                gh[nc_] = gstart(nc_)
        for c in range(NCHK - NBUF, NCHK):
            wh[c].wait()


def kernel(x, index):
    x2d = x.reshape(B * S, D)
    idx_rep = jnp.repeat(index, L)
    out2d = _gather(x2d, idx_rep)
    return out2d.reshape(K, B, D).transpose(1, 0, 2)
